# initial kernel scaffold (unmeasured)
import jax
import jax.numpy as jnp
from jax import lax
from jax.experimental import pallas as pl
from jax.experimental.pallas import tpu as pltpu

N_DEV = 8
_GELU_C = 0.7978845608028654


def kernel(x, w_mat):
    m_per, k_dim = x.shape
    _, n_full = w_mat.shape
    n_per = n_full // N_DEV

    def body(x_ref, w_hbm, out_ref, w_buf, send_buf, recv_buf,
             w_sems, send_sems, recv_sems):
        my = lax.axis_index("i")

        barrier = pltpu.get_barrier_semaphore()
        for d in range(1, N_DEV):
            pl.semaphore_signal(
                barrier, inc=1,
                device_id=((my + d) % N_DEV,),
                device_id_type=pl.DeviceIdType.MESH,
            )
        pl.semaphore_wait(barrier, N_DEV - 1)

        def w_dma(j):
            dst = (my + j) % N_DEV
            return pltpu.make_async_copy(
                w_hbm.at[:, pl.ds(dst * n_per, n_per)],
                w_buf.at[j % 2],
                w_sems.at[j % 2],
            )

        def hop_rdma(j):
            return pltpu.make_async_remote_copy(
                src_ref=send_buf.at[j % 2],
                dst_ref=recv_buf.at[j],
                send_sem=send_sems.at[j],
                recv_sem=recv_sems.at[j],
                device_id=((my + j) % N_DEV,),
                device_id_type=pl.DeviceIdType.MESH,
            )

        w_dma(0).start()
        for j in range(N_DEV):
            w_dma(j).wait()
            if j + 1 < N_DEV:
                w_dma(j + 1).start()
            y = jnp.dot(x_ref[:, :], w_buf[j % 2],
                        preferred_element_type=jnp.float32)
            g = 0.5 * y * (1.0 + jnp.tanh(_GELU_C * (y + 0.044715 * y * y * y)))
            if j == 0:
                out_ref[pl.ds(my * m_per, m_per), :] = g
            else:
                if j >= 3:
                    hop_rdma(j - 2).wait_send()
                send_buf[j % 2] = g.astype(jnp.bfloat16)
                hop_rdma(j).start()

        for j in range(1, N_DEV):
            src = (my - j) % N_DEV
            hop_rdma(j).wait_recv()
            out_ref[pl.ds(src * m_per, m_per), :] = recv_buf[j].astype(jnp.float32)

    return pl.pallas_call(
        body,
        out_shape=jax.ShapeDtypeStruct((N_DEV * m_per, n_per), jnp.float32),
        in_specs=[
            pl.BlockSpec(memory_space=pltpu.VMEM),
            pl.BlockSpec(memory_space=pltpu.ANY),
        ],
        out_specs=pl.BlockSpec(memory_space=pltpu.VMEM),
        scratch_shapes=[
            pltpu.VMEM((2, k_dim, n_per), jnp.bfloat16),
            pltpu.VMEM((2, m_per, n_per), jnp.bfloat16),
            pltpu.VMEM((N_DEV, m_per, n_per), jnp.bfloat16),
            pltpu.SemaphoreType.DMA((2,)),
            pltpu.SemaphoreType.DMA((N_DEV,)),
            pltpu.SemaphoreType.DMA((N_DEV,)),
        ],
        compiler_params=pltpu.CompilerParams(collective_id=0),
    )(x, w_mat)


# baseline (device time: 141201 ns/iter reference)
import jax
import jax.numpy as jnp
from jax import lax
from jax.experimental import pallas as pl
from jax.experimental.pallas import tpu as pltpu

N_DEV = 8
_GELU_C = 0.7978845608028654

_NSLOT = 4


def kernel(x, w_mat):
    m_per, k_dim = x.shape
    _, n_full = w_mat.shape
    n_per = n_full // N_DEV
    kc = 1024
    n_xc = k_dim // n_per
    n_wc = k_dim // kc
    n_task = n_xc + N_DEV * n_wc

    def body(x_hbm, w_hbm, out_hbm, x_bf, stage, out_stage,
             send_buf, recv_buf, stage_sems, out_sems, send_sems, recv_sems):
        my = lax.axis_index("i")

        barrier = pltpu.get_barrier_semaphore()
        for d in range(1, N_DEV):
            pl.semaphore_signal(
                barrier, inc=1,
                device_id=((my + d) % N_DEV,),
                device_id_type=pl.DeviceIdType.MESH,
            )
        pl.semaphore_wait(barrier, N_DEV - 1)

        def in_task(t):
            slot = t % _NSLOT
            if t < n_xc:
                src = x_hbm.at[:, pl.ds(t * n_per, n_per)]
            else:
                u = t - n_xc
                j, c = u // n_wc, u % n_wc
                dst = (my + j) % N_DEV
                src = w_hbm.at[pl.ds(c * kc, kc), pl.ds(dst * n_per, n_per)]
            return pltpu.make_async_copy(src, stage.at[slot], stage_sems.at[slot])

        def consume_x(t):
            x_bf[:, pl.ds(t * n_per, n_per)] = stage[t % _NSLOT].astype(jnp.bfloat16)

        def hop_rdma(j):
            return pltpu.make_async_remote_copy(
                src_ref=send_buf.at[j % 2],
                dst_ref=recv_buf.at[j],
                send_sem=send_sems.at[j],
                recv_sem=recv_sems.at[j],
                device_id=((my + j) % N_DEV,),
                device_id_type=pl.DeviceIdType.MESH,
            )

        out_block = [0]

        def emit_out(row0, val_f32):
            b = out_block[0]
            out_block[0] += 1
            s = b % 2
            if b >= 2:
                pltpu.make_async_copy(
                    out_stage.at[s], out_hbm.at[pl.ds(0, m_per), :],
                    out_sems.at[s],
                ).wait()
            out_stage[s] = val_f32
            pltpu.make_async_copy(
                out_stage.at[s], out_hbm.at[pl.ds(row0, m_per), :],
                out_sems.at[s],
            ).start()

        for t in range(_NSLOT):
            in_task(t).start()

        for t in range(n_xc):
            in_task(t).wait()
            consume_x(t)
            if t + _NSLOT < n_task:
                in_task(t + _NSLOT).start()

        for j in range(N_DEV):
            y = None
            for c in range(n_wc):
                t = n_xc + j * n_wc + c
                in_task(t).wait()
                wv = stage[t % _NSLOT].astype(jnp.bfloat16)
                part = jnp.dot(x_bf[:, pl.ds(c * kc, kc)], wv,
                               preferred_element_type=jnp.float32)
                y = part if y is None else y + part
                if t + _NSLOT < n_task:
                    in_task(t + _NSLOT).start()
            g = 0.5 * y * (1.0 + jnp.tanh(_GELU_C * (y + 0.044715 * y * y * y)))
            if j == 0:
                emit_out(my * m_per, g)
            else:
                if j >= 3:
                    hop_rdma(j - 2).wait_send()
                send_buf[j % 2] = g.astype(jnp.bfloat16)
                hop_rdma(j).start()

        for j in range(1, N_DEV):
            src = (my - j) % N_DEV
            hop_rdma(j).wait_recv()
            emit_out(src * m_per, recv_buf[j].astype(jnp.float32))

        for b in (out_block[0] - 2, out_block[0] - 1):
            pltpu.make_async_copy(
                out_stage.at[b % 2], out_hbm.at[pl.ds(0, m_per), :],
                out_sems.at[b % 2],
            ).wait()
        hop_rdma(N_DEV - 2).wait_send()
        hop_rdma(N_DEV - 1).wait_send()

    return pl.pallas_call(
        body,
        out_shape=jax.ShapeDtypeStruct((N_DEV * m_per, n_per), jnp.float32),
        in_specs=[
            pl.BlockSpec(memory_space=pl.ANY),
            pl.BlockSpec(memory_space=pl.ANY),
        ],
        out_specs=pl.BlockSpec(memory_space=pl.ANY),
        scratch_shapes=[
            pltpu.VMEM((m_per, k_dim), jnp.bfloat16),
            pltpu.VMEM((_NSLOT, m_per, n_per), jnp.float32),
            pltpu.VMEM((2, m_per, n_per), jnp.float32),
            pltpu.VMEM((2, m_per, n_per), jnp.bfloat16),
            pltpu.VMEM((N_DEV, m_per, n_per), jnp.bfloat16),
            pltpu.SemaphoreType.DMA((_NSLOT,)),
            pltpu.SemaphoreType.DMA((2,)),
            pltpu.SemaphoreType.DMA((N_DEV,)),
            pltpu.SemaphoreType.DMA((N_DEV,)),
        ],
        compiler_params=pltpu.CompilerParams(
            collective_id=0,
            vmem_limit_bytes=63 * 1024 * 1024,
        ),
    )(x, w_mat)
